# Initial kernel scaffold; baseline (speedup 1.0000x reference)
import jax
import jax.numpy as jnp
from jax.experimental import pallas as pl


def kernel(x, Wg, We):
    raise NotImplementedError("write your pallas kernel here")


# speedup vs baseline: 3.8854x; 3.8854x over previous
import functools

import jax
import jax.numpy as jnp
from jax.experimental import pallas as pl
from jax.experimental.pallas import tpu as pltpu

_B, _T, _D, _E, _K = 1, 2048, 768, 8, 2
_TM = 256


def _moe_dense_body(x_ref, wg_ref, we_ref, out_ref):
    x = x_ref[...]
    logits = jnp.dot(x, wg_ref[...], preferred_element_type=jnp.float32)
    lane = jax.lax.broadcasted_iota(jnp.int32, logits.shape, 1)
    neg = jnp.float32(-jnp.inf)
    logits = jnp.where(lane < _E, logits, neg)
    m1 = jnp.max(logits, axis=1, keepdims=True)
    i1 = jnp.min(jnp.where(logits == m1, lane, _E), axis=1, keepdims=True)
    l2 = jnp.where(lane == i1, neg, logits)
    m2 = jnp.max(l2, axis=1, keepdims=True)
    i2 = jnp.min(jnp.where(l2 == m2, lane, _E), axis=1, keepdims=True)
    w1 = 1.0 / (1.0 + jnp.exp(m2 - m1))
    w2 = 1.0 - w1
    xb = x.astype(jnp.bfloat16)
    acc = jnp.zeros((_TM, _D), jnp.float32)
    for e in range(_E):
        ye = jnp.dot(xb, we_ref[e], preferred_element_type=jnp.float32)
        wt = jnp.where(i1 == e, w1, 0.0) + jnp.where(i2 == e, w2, 0.0)
        acc = acc + ye * wt
    out_ref[...] = acc


@functools.partial(jax.jit, static_argnames=("interpret",))
def _moe_dense(x2, wgp, web, interpret=False):
    return pl.pallas_call(
        _moe_dense_body,
        grid=(_T // _TM,),
        in_specs=[
            pl.BlockSpec((_TM, _D), lambda i: (i, 0)),
            pl.BlockSpec((_D, 128), lambda i: (0, 0)),
            pl.BlockSpec((_E, _D, _D), lambda i: (0, 0, 0)),
        ],
        out_specs=pl.BlockSpec((_TM, _D), lambda i: (i, 0)),
        out_shape=jax.ShapeDtypeStruct((_T, _D), jnp.float32),
        interpret=interpret,
    )(x2, wgp, web)


def kernel(x, Wg, We):
    x2 = x.reshape(_T, _D)
    wgp = jnp.zeros((_D, 128), jnp.float32).at[:, :_E].set(Wg)
    web = We.astype(jnp.bfloat16)
    out = _moe_dense(x2, wgp, web)
    return out.reshape(_B, _T, _D)
